# baseline (speedup 1.0000x reference)
import functools

import jax
import jax.numpy as jnp
from jax import lax
from jax.experimental import pallas as pl
from jax.experimental.pallas import tpu as pltpu
from jax.experimental.pallas import tpu_sc as plsc

_V = 1_000_000
_E = 64
_B = 4096
_L = 200
_N = _B * _L

_NC = 2
_NS = 16
_NW = _NC * _NS
_PER_W = _N // _NW
_IDXW = 128
_K = 4
_CHUNK = _K * _IDXW
_N_CHUNK = _PER_W // _CHUNK
_IROWS_W = _PER_W // _IDXW

_mesh = plsc.VectorSubcoreMesh(
    core_axis_name="c", subcore_axis_name="s", num_cores=_NC, num_subcores=_NS
)


@functools.partial(
    pl.kernel,
    mesh=_mesh,
    out_type=jax.ShapeDtypeStruct((_N, _E), jnp.float32),
    scratch_types=[
        pltpu.VMEM((_K, _IDXW), jnp.int32),
        pltpu.VMEM((_CHUNK, _E), jnp.float32),
        pltpu.SemaphoreType.DMA,
    ],
    compiler_params=pltpu.CompilerParams(use_tc_tiling_on_sc=False),
)
def _gather_kernel(idx_hbm, table_hbm, out_hbm, idx_v, rows_v, sem):
    wid = lax.axis_index("s") * _NC + lax.axis_index("c")
    irow0 = wid * _IROWS_W
    out0 = wid * _PER_W

    @pl.loop(0, _N_CHUNK)
    def _chunk(g):
        pltpu.sync_copy(idx_hbm.at[pl.ds(irow0 + g * _K, _K)], idx_v)
        copies = [
            pltpu.async_copy(
                table_hbm.at[idx_v.at[j]],
                rows_v.at[pl.ds(j * _IDXW, _IDXW)],
                sem,
            )
            for j in range(_K)
        ]
        for c in copies:
            c.wait()
        pltpu.sync_copy(rows_v, out_hbm.at[pl.ds(out0 + g * _CHUNK, _CHUNK)])


def kernel(x, vocab):
    idx = x.reshape(_N // _IDXW, _IDXW)
    table = vocab.reshape(_V, _E)
    out = _gather_kernel(idx, table)
    return out.reshape(_B, _L, _E)


# speedup vs baseline: 1.0397x; 1.0397x over previous
import functools

import jax
import jax.numpy as jnp
from jax import lax
from jax.experimental import pallas as pl
from jax.experimental.pallas import tpu as pltpu
from jax.experimental.pallas import tpu_sc as plsc

_V = 1_000_000
_E = 64
_B = 4096
_L = 200
_N = _B * _L

_NC = 2
_NS = 16
_NW = _NC * _NS
_PER_W = _N // _NW
_IDXW = 128
_K = 4
_CHUNK = _K * _IDXW
_N_CHUNK = _PER_W // _CHUNK
_PAIRS = _N_CHUNK // 2
_IROWS_W = _PER_W // _IDXW

_mesh = plsc.VectorSubcoreMesh(
    core_axis_name="c", subcore_axis_name="s", num_cores=_NC, num_subcores=_NS
)


@functools.partial(
    pl.kernel,
    mesh=_mesh,
    out_type=jax.ShapeDtypeStruct((_N, _E), jnp.float32),
    scratch_types=[
        pltpu.VMEM((_IROWS_W, _IDXW), jnp.int32),
        pltpu.VMEM((_CHUNK, _E), jnp.float32),
        pltpu.VMEM((_CHUNK, _E), jnp.float32),
        pltpu.SemaphoreType.DMA,
        pltpu.SemaphoreType.DMA,
        pltpu.SemaphoreType.DMA,
        pltpu.SemaphoreType.DMA,
    ],
    compiler_params=pltpu.CompilerParams(use_tc_tiling_on_sc=False),
)
def _gather_kernel(idx_hbm, table_hbm, out_hbm, idx_v, rows0, rows1,
                   sg0, sg1, sw0, sw1):
    wid = lax.axis_index("s") * _NC + lax.axis_index("c")
    irow0 = wid * _IROWS_W
    out0 = wid * _PER_W

    pltpu.sync_copy(idx_hbm.at[pl.ds(irow0, _IROWS_W)], idx_v)

    def fire_gather(chunk, buf, sem):
        return [
            pltpu.async_copy(
                table_hbm.at[idx_v.at[chunk * _K + j]],
                buf.at[pl.ds(j * _IDXW, _IDXW)],
                sem,
            )
            for j in range(_K)
        ]

    def fire_wb(chunk, buf, sem):
        return pltpu.async_copy(
            buf, out_hbm.at[pl.ds(out0 + chunk * _CHUNK, _CHUNK)], sem
        )

    def drain_wb(buf, sem):
        pltpu.make_async_copy(buf, out_hbm.at[pl.ds(out0, _CHUNK)], sem).wait()

    @pl.loop(0, _PAIRS)
    def _pair(p):
        c0 = 2 * p

        @pl.when(p > 0)
        def _():
            drain_wb(rows0, sw0)

        g0 = fire_gather(c0, rows0, sg0)

        @pl.when(p > 0)
        def _():
            drain_wb(rows1, sw1)

        g1 = fire_gather(c0 + 1, rows1, sg1)
        for c in g0:
            c.wait()
        fire_wb(c0, rows0, sw0)
        for c in g1:
            c.wait()
        fire_wb(c0 + 1, rows1, sw1)

    drain_wb(rows0, sw0)
    drain_wb(rows1, sw1)


def kernel(x, vocab):
    idx = x.reshape(_N // _IDXW, _IDXW)
    table = vocab.reshape(_V, _E)
    out = _gather_kernel(idx, table)
    return out.reshape(_B, _L, _E)


# speedup vs baseline: 1.0664x; 1.0257x over previous
import functools

import jax
import jax.numpy as jnp
from jax import lax
from jax.experimental import pallas as pl
from jax.experimental.pallas import tpu as pltpu
from jax.experimental.pallas import tpu_sc as plsc

_V = 1_000_000
_E = 64
_B = 4096
_L = 200

_NC = 2
_NS = 16
_NW = _NC * _NS
_BLK = _B // _NW
_K = 4
_NCHUNK = _L // _K

_mesh = plsc.VectorSubcoreMesh(
    core_axis_name="c", subcore_axis_name="s", num_cores=_NC, num_subcores=_NS
)


@functools.partial(
    pl.kernel,
    mesh=_mesh,
    out_type=jax.ShapeDtypeStruct((_L, _B, _E), jnp.float32),
    scratch_types=[
        pltpu.VMEM((_L, _BLK), jnp.int32),
        pltpu.VMEM((_K, _BLK, _E), jnp.float32),
        pltpu.VMEM((_K, _BLK, _E), jnp.float32),
        pltpu.SemaphoreType.DMA,
        pltpu.SemaphoreType.DMA,
        pltpu.SemaphoreType.DMA,
        pltpu.SemaphoreType.DMA,
    ],
    compiler_params=pltpu.CompilerParams(use_tc_tiling_on_sc=False),
)
def _gather_kernel(xt_hbm, table_hbm, out_hbm, idx_v, rows_a, rows_b,
                   sg_a, sg_b, sw_a, sw_b):
    wid = lax.axis_index("s") * _NC + lax.axis_index("c")
    b0 = wid * _BLK

    pltpu.sync_copy(xt_hbm.at[:, pl.ds(b0, _BLK)], idx_v)

    def fire_gather(chunk, buf, sem):
        return [
            pltpu.async_copy(
                table_hbm.at[idx_v.at[chunk * _K + j]], buf.at[j], sem
            )
            for j in range(_K)
        ]

    def fire_wb(chunk, buf, sem):
        return pltpu.async_copy(
            buf, out_hbm.at[pl.ds(chunk * _K, _K), pl.ds(b0, _BLK)], sem
        )

    def wait_wb(buf, sem):
        pltpu.make_async_copy(
            buf, out_hbm.at[pl.ds(0, _K), pl.ds(b0, _BLK)], sem
        ).wait()

    @pl.loop(0, _NCHUNK // 2)
    def _pair(p):
        c0 = 2 * p

        @pl.when(p > 0)
        def _():
            wait_wb(rows_a, sw_a)

        g0 = fire_gather(c0, rows_a, sg_a)

        @pl.when(p > 0)
        def _():
            wait_wb(rows_b, sw_b)

        g1 = fire_gather(c0 + 1, rows_b, sg_b)
        for c in g0:
            c.wait()
        fire_wb(c0, rows_a, sw_a)
        for c in g1:
            c.wait()
        fire_wb(c0 + 1, rows_b, sw_b)

    wait_wb(rows_a, sw_a)
    wait_wb(rows_b, sw_b)


def kernel(x, vocab):
    table = vocab.reshape(_V, _E)
    out_lbe = _gather_kernel(x.T, table)
    return jnp.transpose(out_lbe, (1, 0, 2))


# speedup vs baseline: 1.0669x; 1.0005x over previous
import functools

import jax
import jax.numpy as jnp
from jax import lax
from jax.experimental import pallas as pl
from jax.experimental.pallas import tpu as pltpu
from jax.experimental.pallas import tpu_sc as plsc

_V = 1_000_000
_E = 64
_B = 4096
_L = 200

_NC = 2
_NS = 16
_NW = _NC * _NS
_BLK = _B // _NW
_LT = _L // 8
_BT = _B // 128
_K = 4
_NCHUNK = _L // _K

_mesh = plsc.VectorSubcoreMesh(
    core_axis_name="c", subcore_axis_name="s", num_cores=_NC, num_subcores=_NS
)


@functools.partial(
    pl.kernel,
    mesh=_mesh,
    out_type=jax.ShapeDtypeStruct((_L, _B, _E), jnp.float32),
    scratch_types=[
        pltpu.VMEM((_LT, 8, _BLK), jnp.int32),
        pltpu.VMEM((_K, _BLK, _E), jnp.float32),
        pltpu.VMEM((_K, _BLK, _E), jnp.float32),
        pltpu.SemaphoreType.DMA,
        pltpu.SemaphoreType.DMA,
        pltpu.SemaphoreType.DMA,
        pltpu.SemaphoreType.DMA,
    ],
    compiler_params=pltpu.CompilerParams(use_tc_tiling_on_sc=False),
)
def _gather_kernel(xt5_hbm, table_hbm, out_hbm, idx_v, rows_a, rows_b,
                   sg_a, sg_b, sw_a, sw_b):
    wid = lax.axis_index("s") * _NC + lax.axis_index("c")
    b0 = wid * _BLK

    pltpu.sync_copy(xt5_hbm.at[:, wid], idx_v)

    def fire_gather(chunk, buf, sem):
        return [
            pltpu.async_copy(
                table_hbm.at[idx_v.at[(chunk * _K + j) // 8,
                                      (chunk * _K + j) % 8]],
                buf.at[j],
                sem,
            )
            for j in range(_K)
        ]

    def fire_wb(chunk, buf, sem):
        return pltpu.async_copy(
            buf, out_hbm.at[pl.ds(chunk * _K, _K), pl.ds(b0, _BLK)], sem
        )

    def wait_wb(buf, sem):
        pltpu.make_async_copy(
            buf, out_hbm.at[pl.ds(0, _K), pl.ds(b0, _BLK)], sem
        ).wait()

    @pl.loop(0, _NCHUNK // 2)
    def _pair(p):
        c0 = 2 * p

        @pl.when(p > 0)
        def _():
            wait_wb(rows_a, sw_a)

        g0 = fire_gather(c0, rows_a, sg_a)

        @pl.when(p > 0)
        def _():
            wait_wb(rows_b, sw_b)

        g1 = fire_gather(c0 + 1, rows_b, sg_b)
        for c in g0:
            c.wait()
        fire_wb(c0, rows_a, sw_a)
        for c in g1:
            c.wait()
        fire_wb(c0 + 1, rows_b, sw_b)

    wait_wb(rows_a, sw_a)
    wait_wb(rows_b, sw_b)


def kernel(x, vocab):
    table = vocab.reshape(_V, _E)
    xt5 = x.T.reshape(_LT, 8, _BT, 128).transpose(0, 2, 1, 3)
    out_lbe = _gather_kernel(xt5, table)
    return jnp.transpose(out_lbe, (1, 0, 2))
